# Initial kernel scaffold; baseline (speedup 1.0000x reference)
import jax
import jax.numpy as jnp
from jax.experimental import pallas as pl


def kernel(x, w_enc, bn_gamma, bn_beta, emb, w_dec):
    raise NotImplementedError("write your pallas kernel here")


# speedup vs baseline: 1.2929x; 1.2929x over previous
import jax
import jax.numpy as jnp
from jax.experimental import pallas as pl
import jax.experimental.pallas.tpu as pltpu

_B, _H, _W = 4, 8, 128
_C = 16
_K = 1024
_D = 128
_TK = 15
_PAD = 7
_EPS = 1e-5
_V = _B * _C * _H
_KB = 16
_STEPS = _K // _KB


def _vq_kernel(ze_ref, eblk_ref, emb_ref, wcol_ref, xt_ref, zq_ref,
               best_ref, bidx_ref, pad_ref):
    step = pl.program_id(0)

    @pl.when(step == 0)
    def _init():
        best_ref[...] = jnp.full((_V, 1), jnp.inf, jnp.float32)
        bidx_ref[...] = jnp.zeros((_V, 1), jnp.int32)

    v = ze_ref[...].reshape(_V, _D)
    for j in range(_KB):
        diff = v - eblk_ref[j:j + 1, :]
        dcol = jnp.sum(diff * diff, axis=1, keepdims=True)
        upd = dcol < best_ref[...]
        best_ref[...] = jnp.where(upd, dcol, best_ref[...])
        bidx_ref[...] = jnp.where(upd, step * _KB + j, bidx_ref[...])

    @pl.when(step == _STEPS - 1)
    def _finish():
        idx = bidx_ref[...]
        lane = jax.lax.broadcasted_iota(jnp.int32, (_V, _K), 1)
        onehot = (lane == idx).astype(jnp.float32)
        zq = jax.lax.dot_general(onehot, emb_ref[...], (((1,), (0,)), ((), ())),
                                 precision=jax.lax.Precision.HIGHEST,
                                 preferred_element_type=jnp.float32)
        zq_ref[...] = zq
        pad_ref[...] = jnp.zeros((_V, 2 * _W), jnp.float32)
        pad_ref[:, 0:_W] = zq
        wide = pad_ref[...]
        dec = jnp.zeros((_V, _W), jnp.float32)
        for t in range(_TK):
            shifted = pltpu.roll(wide, (_PAD - t) % (2 * _W), axis=1)[:, 0:_W]
            dec = dec + shifted * wcol_ref[t]
        for b in range(_B):
            blk = jnp.zeros((_H, _W), jnp.float32)
            for c in range(_C):
                blk = blk + dec[b * _C * _H + c * _H:b * _C * _H + c * _H + _H, :]
            xt_ref[b * _H:(b + 1) * _H, :] = jnp.tanh(blk)


def kernel(x, w_enc, bn_gamma, bn_beta, emb, w_dec):
    z = jax.lax.conv_general_dilated(
        x, w_enc, window_strides=(1, 1), padding=((0, 0), (_PAD, _PAD)),
        dimension_numbers=("NCHW", "OIHW", "NCHW"))
    mean = jnp.mean(z, axis=(0, 2, 3), keepdims=True)
    var = jnp.var(z, axis=(0, 2, 3), keepdims=True)
    z_e_x = (z - mean) / jnp.sqrt(var + _EPS)
    z_e_x = z_e_x * bn_gamma.reshape(1, -1, 1, 1) + bn_beta.reshape(1, -1, 1, 1)

    wt = w_dec[:, 0, 0, ::-1]
    c_idx = (jnp.arange(_V) // _H) % _C
    wcols = wt[c_idx].T[:, :, None]

    xt, zq = pl.pallas_call(
        _vq_kernel,
        grid=(_STEPS,),
        in_specs=[
            pl.BlockSpec((_B, _C, _H, _W), lambda s: (0, 0, 0, 0)),
            pl.BlockSpec((_KB, _D), lambda s: (s, 0)),
            pl.BlockSpec((_K, _D), lambda s: (0, 0)),
            pl.BlockSpec((_TK, _V, 1), lambda s: (0, 0, 0)),
        ],
        out_specs=(
            pl.BlockSpec((_B * _H, _W), lambda s: (0, 0)),
            pl.BlockSpec((_V, _D), lambda s: (0, 0)),
        ),
        out_shape=(
            jax.ShapeDtypeStruct((_B * _H, _W), jnp.float32),
            jax.ShapeDtypeStruct((_V, _D), jnp.float32),
        ),
        scratch_shapes=[
            pltpu.VMEM((_V, 1), jnp.float32),
            pltpu.VMEM((_V, 1), jnp.int32),
            pltpu.VMEM((_V, 2 * _W), jnp.float32),
        ],
    )(z_e_x, emb, emb, wcols)

    x_tilde = xt.reshape(_B, 1, _H, _W)
    z_q_x = zq.reshape(_B, _C, _H, _D)
    return (x_tilde, z_e_x, z_q_x)


# speedup vs baseline: 1.3681x; 1.0582x over previous
import jax
import jax.numpy as jnp
from jax.experimental import pallas as pl
import jax.experimental.pallas.tpu as pltpu

_B, _H, _W = 4, 8, 128
_C = 16
_K = 1024
_D = 128
_TK = 15
_PAD = 7
_EPS = 1e-5
_V = _B * _C * _H
_KB = 64
_STEPS = _K // _KB


def _vq_kernel(ze_ref, eblk_ref, emb_ref, wcol_ref, xt_ref, zq_ref,
               best_ref, bidx_ref, pad_ref):
    step = pl.program_id(0)

    @pl.when(step == 0)
    def _init():
        best_ref[...] = jnp.full((_V, 1), jnp.inf, jnp.float32)
        bidx_ref[...] = jnp.zeros((_V, 1), jnp.int32)

    v = ze_ref[...].reshape(_V, _D)
    for j in range(_KB):
        diff = v - eblk_ref[j:j + 1, :]
        dcol = jnp.sum(diff * diff, axis=1, keepdims=True)
        upd = dcol < best_ref[...]
        best_ref[...] = jnp.where(upd, dcol, best_ref[...])
        bidx_ref[...] = jnp.where(upd, step * _KB + j, bidx_ref[...])

    @pl.when(step == _STEPS - 1)
    def _finish():
        idx = bidx_ref[...]
        lane = jax.lax.broadcasted_iota(jnp.int32, (_V, _K), 1)
        onehot = (lane == idx).astype(jnp.float32)
        zq = jax.lax.dot_general(onehot, emb_ref[...], (((1,), (0,)), ((), ())),
                                 precision=jax.lax.Precision.HIGHEST,
                                 preferred_element_type=jnp.float32)
        zq_ref[...] = zq
        pad_ref[...] = jnp.zeros((_V, 2 * _W), jnp.float32)
        pad_ref[:, 0:_W] = zq
        wide = pad_ref[...]
        dec = jnp.zeros((_V, _W), jnp.float32)
        for t in range(_TK):
            shifted = pltpu.roll(wide, (_PAD - t) % (2 * _W), axis=1)[:, 0:_W]
            dec = dec + shifted * wcol_ref[t]
        for b in range(_B):
            blk = jnp.zeros((_H, _W), jnp.float32)
            for c in range(_C):
                blk = blk + dec[b * _C * _H + c * _H:b * _C * _H + c * _H + _H, :]
            xt_ref[b * _H:(b + 1) * _H, :] = jnp.tanh(blk)


def kernel(x, w_enc, bn_gamma, bn_beta, emb, w_dec):
    z = jax.lax.conv_general_dilated(
        x, w_enc, window_strides=(1, 1), padding=((0, 0), (_PAD, _PAD)),
        dimension_numbers=("NCHW", "OIHW", "NCHW"))
    mean = jnp.mean(z, axis=(0, 2, 3), keepdims=True)
    var = jnp.var(z, axis=(0, 2, 3), keepdims=True)
    z_e_x = (z - mean) / jnp.sqrt(var + _EPS)
    z_e_x = z_e_x * bn_gamma.reshape(1, -1, 1, 1) + bn_beta.reshape(1, -1, 1, 1)

    wt = w_dec[:, 0, 0, ::-1]
    c_idx = (jnp.arange(_V) // _H) % _C
    wcols = wt[c_idx].T[:, :, None]

    xt, zq = pl.pallas_call(
        _vq_kernel,
        grid=(_STEPS,),
        in_specs=[
            pl.BlockSpec((_B, _C, _H, _W), lambda s: (0, 0, 0, 0)),
            pl.BlockSpec((_KB, _D), lambda s: (s, 0)),
            pl.BlockSpec((_K, _D), lambda s: (0, 0)),
            pl.BlockSpec((_TK, _V, 1), lambda s: (0, 0, 0)),
        ],
        out_specs=(
            pl.BlockSpec((_B * _H, _W), lambda s: (0, 0)),
            pl.BlockSpec((_V, _D), lambda s: (0, 0)),
        ),
        out_shape=(
            jax.ShapeDtypeStruct((_B * _H, _W), jnp.float32),
            jax.ShapeDtypeStruct((_V, _D), jnp.float32),
        ),
        scratch_shapes=[
            pltpu.VMEM((_V, 1), jnp.float32),
            pltpu.VMEM((_V, 1), jnp.int32),
            pltpu.VMEM((_V, 2 * _W), jnp.float32),
        ],
    )(z_e_x, emb, emb, wcols)

    x_tilde = xt.reshape(_B, 1, _H, _W)
    z_q_x = zq.reshape(_B, _C, _H, _D)
    return (x_tilde, z_e_x, z_q_x)


# speedup vs baseline: 1.3855x; 1.0128x over previous
import jax
import jax.numpy as jnp
from jax.experimental import pallas as pl
import jax.experimental.pallas.tpu as pltpu

_B, _H, _W = 4, 8, 128
_C = 16
_K = 1024
_D = 128
_TK = 15
_PAD = 7
_EPS = 1e-5
_V = _B * _C * _H
_KB = 256
_STEPS = _K // _KB


def _vq_kernel(ze_ref, eblk_ref, emb_ref, wcol_ref, xt_ref, zq_ref,
               best_ref, bidx_ref, pad_ref):
    step = pl.program_id(0)

    @pl.when(step == 0)
    def _init():
        best_ref[...] = jnp.full((_V, 1), jnp.inf, jnp.float32)
        bidx_ref[...] = jnp.zeros((_V, 1), jnp.int32)

    v = ze_ref[...].reshape(_V, _D)
    for j in range(_KB):
        diff = v - eblk_ref[j:j + 1, :]
        dcol = jnp.sum(diff * diff, axis=1, keepdims=True)
        upd = dcol < best_ref[...]
        best_ref[...] = jnp.where(upd, dcol, best_ref[...])
        bidx_ref[...] = jnp.where(upd, step * _KB + j, bidx_ref[...])

    @pl.when(step == _STEPS - 1)
    def _finish():
        idx = bidx_ref[...]
        lane = jax.lax.broadcasted_iota(jnp.int32, (_V, _K), 1)
        onehot = (lane == idx).astype(jnp.float32)
        zq = jax.lax.dot_general(onehot, emb_ref[...], (((1,), (0,)), ((), ())),
                                 precision=jax.lax.Precision.HIGHEST,
                                 preferred_element_type=jnp.float32)
        zq_ref[...] = zq
        pad_ref[...] = jnp.zeros((_V, 2 * _W), jnp.float32)
        pad_ref[:, 0:_W] = zq
        wide = pad_ref[...]
        dec = jnp.zeros((_V, _W), jnp.float32)
        for t in range(_TK):
            shifted = pltpu.roll(wide, (_PAD - t) % (2 * _W), axis=1)[:, 0:_W]
            dec = dec + shifted * wcol_ref[t]
        for b in range(_B):
            blk = jnp.zeros((_H, _W), jnp.float32)
            for c in range(_C):
                blk = blk + dec[b * _C * _H + c * _H:b * _C * _H + c * _H + _H, :]
            xt_ref[b * _H:(b + 1) * _H, :] = jnp.tanh(blk)


def kernel(x, w_enc, bn_gamma, bn_beta, emb, w_dec):
    z = jax.lax.conv_general_dilated(
        x, w_enc, window_strides=(1, 1), padding=((0, 0), (_PAD, _PAD)),
        dimension_numbers=("NCHW", "OIHW", "NCHW"))
    mean = jnp.mean(z, axis=(0, 2, 3), keepdims=True)
    var = jnp.var(z, axis=(0, 2, 3), keepdims=True)
    z_e_x = (z - mean) / jnp.sqrt(var + _EPS)
    z_e_x = z_e_x * bn_gamma.reshape(1, -1, 1, 1) + bn_beta.reshape(1, -1, 1, 1)

    wt = w_dec[:, 0, 0, ::-1]
    c_idx = (jnp.arange(_V) // _H) % _C
    wcols = wt[c_idx].T[:, :, None]

    xt, zq = pl.pallas_call(
        _vq_kernel,
        grid=(_STEPS,),
        in_specs=[
            pl.BlockSpec((_B, _C, _H, _W), lambda s: (0, 0, 0, 0)),
            pl.BlockSpec((_KB, _D), lambda s: (s, 0)),
            pl.BlockSpec((_K, _D), lambda s: (0, 0)),
            pl.BlockSpec((_TK, _V, 1), lambda s: (0, 0, 0)),
        ],
        out_specs=(
            pl.BlockSpec((_B * _H, _W), lambda s: (0, 0)),
            pl.BlockSpec((_V, _D), lambda s: (0, 0)),
        ),
        out_shape=(
            jax.ShapeDtypeStruct((_B * _H, _W), jnp.float32),
            jax.ShapeDtypeStruct((_V, _D), jnp.float32),
        ),
        scratch_shapes=[
            pltpu.VMEM((_V, 1), jnp.float32),
            pltpu.VMEM((_V, 1), jnp.int32),
            pltpu.VMEM((_V, 2 * _W), jnp.float32),
        ],
    )(z_e_x, emb, emb, wcols)

    x_tilde = xt.reshape(_B, 1, _H, _W)
    z_q_x = zq.reshape(_B, _C, _H, _D)
    return (x_tilde, z_e_x, z_q_x)
